# baseline (speedup 1.0000x reference)
import functools

import jax
import jax.numpy as jnp
from jax import lax
from jax.experimental import pallas as pl
from jax.experimental.pallas import tpu as pltpu
from jax.experimental.pallas import tpu_sc as plsc

N = 50000
E = 800000
D = 64
Q = 8192

CH = 128
EB = 2048
TCH = E // CH
QN = 8448
NREG = 6
DEAD = QN
AR = QN + CH
NP = 51200
NPT = NP // 16

QPT = Q // 32


def _mesh():
    return plsc.VectorSubcoreMesh(core_axis_name="c", subcore_axis_name="s")


def _chunk_range(s):
    n = 390 + (s < 10).astype(jnp.int32)
    base = s * 390 + jnp.minimum(s, 10)
    return base, n


@functools.cache
def _make_segsum():
    @functools.partial(
        pl.kernel,
        mesh=_mesh(),
        compiler_params=pltpu.CompilerParams(needs_layout_passes=False),
        out_type=jax.ShapeDtypeStruct((NREG * QN, 128), jnp.float32),
        scratch_types=[
            pltpu.VMEM((EB,), jnp.int32),
            pltpu.VMEM((EB,), jnp.int32),
            pltpu.VMEM((2 * CH,), jnp.int32),
            pltpu.VMEM((2 * CH,), jnp.int32),
            pltpu.VMEM((2 * CH,), jnp.int32),
            pltpu.VMEM((2 * CH,), jnp.int32),
            pltpu.VMEM((2 * CH,), jnp.int32),
            pltpu.VMEM((2 * CH,), jnp.int32),
            pltpu.VMEM((CH, 128), jnp.float32),
            pltpu.VMEM((CH, 128), jnp.float32),
            pltpu.VMEM((CH, 128), jnp.float32),
            pltpu.VMEM_SHARED((AR, 128), jnp.float32),
            pltpu.SemaphoreType.DMA,
            pltpu.SemaphoreType.DMA,
            pltpu.SemaphoreType.DMA,
            pltpu.SemaphoreType.DMA,
            pltpu.SemaphoreType.DMA,
            pltpu.SemaphoreType.DMA,
        ],
    )
    def _segsum_sc(table, src_g, dst_g, out, sv, dv, csrcA, cdstA, csrcB,
                   cdstB, csrcC, cdstC, rowsA, rowsB, rowsC, acc,
                   gsA, gsB, gsC, ssA, ssB, ssC):
        c = lax.axis_index("c")
        s = lax.axis_index("s")

        z16 = jnp.zeros((16,), jnp.float32)
        pos16 = lax.broadcasted_iota(jnp.int32, (16,), 0)

        nz = 4 + (s < 3).astype(jnp.int32)
        nw = 4 + (s < 2).astype(jnp.int32)
        nb = 24 + (s < 6).astype(jnp.int32)

        pairs = ((csrcA, cdstA, rowsA, gsA, ssA),
                 (csrcB, cdstB, rowsB, gsB, ssB),
                 (csrcC, cdstC, rowsC, gsC, ssC))
        first = pl.ds(0, CH)

        def _wait_gather(pair):
            csrc_o, _, rows_o, gs_o, _ = pair
            pltpu.make_async_copy(table.at[csrc_o.at[first]], rows_o,
                                  gs_o).wait()

        def _fire_scatter(pair):
            _, cdst_o, rows_o, _, ss_o = pair
            pltpu.async_copy(rows_o, acc.at[cdst_o.at[first]], ss_o,
                             add=True)

        def _wait_scatter(pair):
            _, cdst_o, rows_o, _, ss_o = pair
            pltpu.make_async_copy(rows_o, acc.at[cdst_o.at[first]],
                                  ss_o).wait()

        def _sync_drain(pair):
            _, cdst_o, rows_o, _, _ = pair
            _wait_gather(pair)
            pltpu.sync_copy(rows_o, acc.at[cdst_o.at[first]], add=True)

        def _switch3(m, f0, f1, f2):
            lax.cond(m == 0, f0, lambda: lax.cond(m == 1, f1, f2))

        def _pass(p, _):
            q = 2 * p + c
            qbase = q * QN

            def _zb(i, _):
                rowsA[i // 8, pl.ds((i % 8) * 16, 16)] = z16
                return 0
            lax.fori_loop(0, CH * 8, _zb, 0)

            def _z(k, _):
                pltpu.sync_copy(rowsA, acc.at[pl.ds((s + 16 * k) * CH, CH)])
                return 0
            lax.fori_loop(0, nz, _z, 0)
            plsc.subcore_barrier()

            def _sub_for(i):
                cur = pairs[i]
                prev = pairs[(i - 1) % 3]
                nxt = pairs[(i + 1) % 3]
                csrc_c, cdst_c, rows_c, gs_c, _ = cur

                def _sub_impl(state, sb):
                    off, fc = state
                    for v in range(CH // 16):
                        sl = pl.ds(sb + v * 16, 16)
                        local = dv[sl] - qbase
                        m = (local >= 0) & (local < QN)
                        plsc.store_compressed(csrc_c.at[pl.ds(off, 16)],
                                              sv[sl], mask=m)
                        plsc.store_compressed(cdst_c.at[pl.ds(off, 16)],
                                              local, mask=m)
                        off = off + jnp.sum(m.astype(jnp.int32))

                    def _flush(state):
                        off, fc = state
                        pltpu.async_copy(table.at[csrc_c.at[first]], rows_c,
                                         gs_c)

                        @pl.when(fc >= 1)
                        def _():
                            _sync_drain(prev)
                        csrc_n, cdst_n, _, _, _ = nxt
                        for i2 in range(CH // 16):
                            t0 = csrc_c[pl.ds(CH + i2 * 16, 16)]
                            csrc_n[pl.ds(i2 * 16, 16)] = t0
                            t1 = cdst_c[pl.ds(CH + i2 * 16, 16)]
                            cdst_n[pl.ds(i2 * 16, 16)] = t1
                        return off - CH, fc + 1

                    return lax.cond(off >= CH, _flush, lambda st: st,
                                    (off, fc))
                return _sub_impl

            _subs = tuple(_sub_for(i) for i in range(3))

            def _sub(state, sb):
                off, fc = state
                m = fc % 3
                return lax.cond(
                    m == 0, lambda st: _subs[0](st, sb),
                    lambda st: lax.cond(
                        m == 1, lambda s2: _subs[1](s2, sb),
                        lambda s2: _subs[2](s2, sb), st),
                    (off, fc))

            def _step(k, state):
                eb = (s + 16 * k) * EB
                pltpu.sync_copy(src_g.at[pl.ds(eb, EB)], sv)
                pltpu.sync_copy(dst_g.at[pl.ds(eb, EB)], dv)
                return lax.fori_loop(
                    0, EB // CH, lambda j, st: _sub(st, j * CH), state)
            state = lax.fori_loop(0, nb, _step,
                                  (jnp.int32(0), jnp.int32(0)))

            def _tail(state):
                tb = 390 * EB + s * CH
                pltpu.sync_copy(src_g.at[pl.ds(tb, CH)], sv.at[pl.ds(0, CH)])
                pltpu.sync_copy(dst_g.at[pl.ds(tb, CH)], dv.at[pl.ds(0, CH)])
                return _sub(state, 0)
            off, fc = lax.cond(s < 10, _tail, lambda st: st, state)

            @pl.when(fc >= 1)
            def _():
                m = (fc - 1) % 3
                _switch3(m, lambda: _sync_drain(pairs[0]),
                         lambda: _sync_drain(pairs[1]),
                         lambda: _sync_drain(pairs[2]))

            def _finish_for(cur):
                csrc_c, cdst_c, rows_c, gs_c, _ = cur

                def _go():
                    for v in range(CH // 16):
                        sl = pl.ds(v * 16, 16)
                        keep = (pos16 + v * 16) < off
                        csrc_c[sl] = jnp.where(keep, csrc_c[sl], 0)
                        cdst_c[sl] = jnp.where(keep, cdst_c[sl], DEAD)
                    pltpu.async_copy(table.at[csrc_c.at[first]], rows_c,
                                     gs_c).wait()
                    pltpu.sync_copy(rows_c, acc.at[cdst_c.at[first]],
                                    add=True)
                return _go

            @pl.when(off > 0)
            def _():
                _switch3(fc % 3, _finish_for(pairs[0]),
                         _finish_for(pairs[1]), _finish_for(pairs[2]))
            plsc.subcore_barrier()

            def _w(k, _):
                ch = s + 16 * k
                pltpu.sync_copy(acc.at[pl.ds(ch * CH, CH)],
                                out.at[pl.ds(qbase + ch * CH, CH)])
                return 0
            lax.fori_loop(0, nw, _w, 0)
            plsc.subcore_barrier()
            return 0
        lax.fori_loop(0, NREG // 2, _pass, 0)

    return _segsum_sc


@functools.cache
def _make_counts():
    @functools.partial(
        pl.kernel,
        mesh=_mesh(),
        compiler_params=pltpu.CompilerParams(needs_layout_passes=False),
        out_type=[
            jax.ShapeDtypeStruct((NP,), jnp.float32),
            jax.ShapeDtypeStruct((NP,), jnp.float32),
        ],
        scratch_types=[
            pltpu.VMEM((NP,), jnp.float32),
            pltpu.VMEM((EB,), jnp.int32),
            pltpu.VMEM((NPT,), jnp.float32),
            pltpu.VMEM((NPT,), jnp.float32),
            pltpu.VMEM_SHARED((16 * NP,), jnp.float32),
            pltpu.SemaphoreType.DMA,
        ],
    )
    def _counts_sc(dst_a, dst_b, out_a, out_b, acc, dblk, rbuf, res,
                   shared, sem):
        c = lax.axis_index("c")
        s = lax.axis_index("s")

        z16 = jnp.zeros((16,), jnp.float32)
        ones16 = jnp.ones((16,), jnp.float32)

        def _z(i, _):
            acc[pl.ds(i * 16, 16)] = z16
            return 0
        lax.fori_loop(0, NP // 16, _z, 0)

        nb = 24 + (s < 6).astype(jnp.int32)

        def _hist(dref):
            def _blk(b, _):
                pltpu.sync_copy(dref.at[pl.ds((s + 16 * b) * EB, EB)], dblk)

                def _vec(v, _):
                    idx = dblk[pl.ds(v * 16, 16)]
                    plsc.addupdate_scatter(acc, [idx], ones16)
                    return 0
                lax.fori_loop(0, EB // 16, _vec, 0)
                return 0
            lax.fori_loop(0, nb, _blk, 0)

            @pl.when(s < 10)
            def _():
                tb = 390 * EB + s * CH
                pltpu.sync_copy(dref.at[pl.ds(tb, CH)], dblk.at[pl.ds(0, CH)])

                def _vec(v, _):
                    idx = dblk[pl.ds(v * 16, 16)]
                    plsc.addupdate_scatter(acc, [idx], ones16)
                    return 0
                lax.fori_loop(0, CH // 16, _vec, 0)

        @pl.when(c == 0)
        def _():
            _hist(dst_a)

        @pl.when(c == 1)
        def _():
            _hist(dst_b)

        pltpu.sync_copy(acc, shared.at[pl.ds(s * NP, NP)])
        plsc.subcore_barrier()

        rb = s * NPT
        pltpu.sync_copy(shared.at[pl.ds(rb, NPT)], res)

        def _red(t, _):
            pltpu.sync_copy(shared.at[pl.ds(t * NP + rb, NPT)], rbuf)

            def _add(v, _):
                sl = pl.ds(v * 16, 16)
                res[sl] = res[sl] + rbuf[sl]
                return 0
            lax.fori_loop(0, NPT // 16, _add, 0)
            return 0
        lax.fori_loop(1, 16, _red, 0)

        @pl.when(c == 0)
        def _():
            pltpu.sync_copy(res, out_a.at[pl.ds(rb, NPT)])

        @pl.when(c == 1)
        def _():
            pltpu.sync_copy(res, out_b.at[pl.ds(rb, NPT)])

    return _counts_sc


@functools.cache
def _make_head():
    @functools.partial(
        pl.kernel,
        mesh=_mesh(),
        compiler_params=pltpu.CompilerParams(needs_layout_passes=False),
        out_type=jax.ShapeDtypeStruct((Q,), jnp.float32),
        scratch_types=[
            pltpu.VMEM((N,), jnp.float32),
            pltpu.VMEM((N,), jnp.float32),
            pltpu.VMEM((QPT,), jnp.int32),
            pltpu.VMEM((QPT,), jnp.int32),
            pltpu.VMEM((QPT,), jnp.float32),
            pltpu.SemaphoreType.DMA,
        ],
    )
    def _head_sc(p_user, p_item, eli0, eli1, out, pu, pi, i0, i1, acc, sem):
        c = lax.axis_index("c")
        s = lax.axis_index("s")
        wid = s * 2 + c
        base = wid * QPT
        pltpu.sync_copy(p_user, pu)
        pltpu.sync_copy(p_item, pi)
        pltpu.sync_copy(eli0.at[pl.ds(base, QPT)], i0)
        pltpu.sync_copy(eli1.at[pl.ds(base, QPT)], i1)

        def _g(v, _):
            sl = pl.ds(v * 16, 16)
            a = plsc.load_gather(pu, [i0[sl]])
            b = plsc.load_gather(pi, [i1[sl]])
            acc[sl] = a + b
            return 0
        lax.fori_loop(0, QPT // 16, _g, 0)
        pltpu.sync_copy(acc, out.at[pl.ds(base, QPT)])

    return _head_sc


BR = 2000
NB = N // BR


def _mm_pad(x, W):
    def body(x_ref, w_ref, o_ref):
        y = jnp.dot(x_ref[...], w_ref[...], preferred_element_type=jnp.float32)
        o_ref[...] = jnp.concatenate([y, jnp.zeros((BR, 128 - D), jnp.float32)],
                                     axis=1)
    return pl.pallas_call(
        body,
        grid=(NB,),
        in_specs=[pl.BlockSpec((BR, D), lambda i: (i, 0)),
                  pl.BlockSpec((D, D), lambda i: (0, 0))],
        out_specs=pl.BlockSpec((BR, 128), lambda i: (i, 0)),
        out_shape=jax.ShapeDtypeStruct((N, 128), jnp.float32),
    )(x, W)


def _combine(xd, W_self, b, S, cnt):
    def body(x_ref, w_ref, b_ref, s_ref, c_ref, h_ref, st_ref, acc):
        i = pl.program_id(0)
        h = (jnp.dot(x_ref[...], w_ref[...], preferred_element_type=jnp.float32)
             + s_ref[:, :D] / jnp.maximum(c_ref[...], 1.0) + b_ref[...])
        h_ref[...] = h

        @pl.when(i == 0)
        def _():
            acc[...] = jnp.zeros_like(acc)

        acc[0:1, :] += jnp.sum(h, axis=0, keepdims=True)
        acc[1:2, :] += jnp.sum(h * h, axis=0, keepdims=True)
        st_ref[...] = acc[...]

    return pl.pallas_call(
        body,
        grid=(NB,),
        in_specs=[pl.BlockSpec((BR, D), lambda i: (i, 0)),
                  pl.BlockSpec((D, D), lambda i: (0, 0)),
                  pl.BlockSpec((1, D), lambda i: (0, 0)),
                  pl.BlockSpec((BR, 128), lambda i: (i, 0)),
                  pl.BlockSpec((BR, 1), lambda i: (i, 0))],
        out_specs=[pl.BlockSpec((BR, D), lambda i: (i, 0)),
                   pl.BlockSpec((8, D), lambda i: (0, 0))],
        out_shape=[jax.ShapeDtypeStruct((N, D), jnp.float32),
                   jax.ShapeDtypeStruct((8, D), jnp.float32)],
        scratch_shapes=[pltpu.VMEM((8, D), jnp.float32)],
    )(xd, W_self, b, S, cnt)


def _bn_act(h, st, g, b, leaky):
    def body(h_ref, st_ref, g_ref, b_ref, o_ref):
        mu = st_ref[0:1, :] / N
        var = st_ref[1:2, :] / N - mu * mu
        o = (h_ref[...] - mu) / jnp.sqrt(var + 1e-5) * g_ref[...] + b_ref[...]
        if leaky:
            o = jnp.where(o >= 0, o, 0.01 * o)
        o_ref[...] = o

    return pl.pallas_call(
        body,
        grid=(NB,),
        in_specs=[pl.BlockSpec((BR, D), lambda i: (i, 0)),
                  pl.BlockSpec((8, D), lambda i: (0, 0)),
                  pl.BlockSpec((1, D), lambda i: (0, 0)),
                  pl.BlockSpec((1, D), lambda i: (0, 0))],
        out_specs=pl.BlockSpec((BR, D), lambda i: (i, 0)),
        out_shape=jax.ShapeDtypeStruct((N, D), jnp.float32),
    )(h, st, g, b)


def _bn_proj(h, st, g, b, w, bias):
    def body(h_ref, st_ref, g_ref, b_ref, w_ref, bias_ref, o_ref):
        mu = st_ref[0:1, :] / N
        var = st_ref[1:2, :] / N - mu * mu
        o = (h_ref[...] - mu) / jnp.sqrt(var + 1e-5) * g_ref[...] + b_ref[...]
        o_ref[...] = jnp.dot(o, w_ref[...],
                             preferred_element_type=jnp.float32) + bias_ref[...]

    return pl.pallas_call(
        body,
        grid=(NB,),
        in_specs=[pl.BlockSpec((BR, D), lambda i: (i, 0)),
                  pl.BlockSpec((8, D), lambda i: (0, 0)),
                  pl.BlockSpec((1, D), lambda i: (0, 0)),
                  pl.BlockSpec((1, D), lambda i: (0, 0)),
                  pl.BlockSpec((D, 1), lambda i: (0, 0)),
                  pl.BlockSpec((1, 1), lambda i: (0, 0))],
        out_specs=pl.BlockSpec((BR, 1), lambda i: (i, 0)),
        out_shape=jax.ShapeDtypeStruct((N, 1), jnp.float32),
    )(h, st, g, b, w, bias)


def kernel(x_user, x_item, edge_index_u2i, edge_index_i2u, edge_label_index_u2i,
           W_self1_u2i, W_neigh1_u2i, b1_u2i, W_self1_i2u, W_neigh1_i2u, b1_i2u,
           gamma1_user, beta1_user, gamma1_item, beta1_item,
           W_self2_u2i, W_neigh2_u2i, b2_u2i, W_self2_i2u, W_neigh2_i2u, b2_i2u,
           gamma2_user, beta2_user, gamma2_item, beta2_item,
           W_mlp, b_mlp):
    segsum = _make_segsum()
    counts = _make_counts()
    head = _make_head()

    src_u2i = edge_index_u2i[0]
    dst_u2i = edge_index_u2i[1]
    src_i2u = edge_index_i2u[0]
    dst_i2u = edge_index_i2u[1]

    cnt_item, cnt_user = counts(dst_u2i, dst_i2u)
    cnt_item = cnt_item[:N, None]
    cnt_user = cnt_user[:N, None]

    def sage(x_src, x_dst, W_neigh, W_self, b, src, dst, cnt):
        S = segsum(_mm_pad(x_src, W_neigh), src, dst)
        return _combine(x_dst, W_self, b.reshape(1, D), S[:N], cnt)

    h_item, st_i = sage(x_user, x_item, W_neigh1_u2i, W_self1_u2i, b1_u2i,
                        src_u2i, dst_u2i, cnt_item)
    h_user, st_u = sage(x_item, x_user, W_neigh1_i2u, W_self1_i2u, b1_i2u,
                        src_i2u, dst_i2u, cnt_user)

    act_user = _bn_act(h_user, st_u, gamma1_user.reshape(1, D),
                       beta1_user.reshape(1, D), leaky=True)
    act_item = _bn_act(h_item, st_i, gamma1_item.reshape(1, D),
                       beta1_item.reshape(1, D), leaky=True)

    h_item2, st_i2 = sage(act_user, act_item, W_neigh2_u2i, W_self2_u2i,
                          b2_u2i, src_u2i, dst_u2i, cnt_item)
    h_user2, st_u2 = sage(act_item, act_user, W_neigh2_i2u, W_self2_i2u,
                          b2_i2u, src_i2u, dst_i2u, cnt_user)

    p_user = _bn_proj(h_user2, st_u2, gamma2_user.reshape(1, D),
                      beta2_user.reshape(1, D), W_mlp[:D], b_mlp.reshape(1, 1))
    p_item = _bn_proj(h_item2, st_i2, gamma2_item.reshape(1, D),
                      beta2_item.reshape(1, D), W_mlp[D:],
                      jnp.zeros((1, 1), jnp.float32))

    pred = head(p_user.reshape(N), p_item.reshape(N),
                edge_label_index_u2i[0], edge_label_index_u2i[1])
    return pred


# speedup vs baseline: 1.0909x; 1.0909x over previous
import functools

import jax
import jax.numpy as jnp
from jax import lax
from jax.experimental import pallas as pl
from jax.experimental.pallas import tpu as pltpu
from jax.experimental.pallas import tpu_sc as plsc

N = 50000
E = 800000
D = 64
Q = 8192

CH = 128
EB = 2048
TCH = E // CH
QN = 8448
NREG = 6
DEAD = QN
AR = QN + CH
NP = 51200
NPT = NP // 16

QPT = Q // 32


def _mesh():
    return plsc.VectorSubcoreMesh(core_axis_name="c", subcore_axis_name="s")


def _chunk_range(s):
    n = 390 + (s < 10).astype(jnp.int32)
    base = s * 390 + jnp.minimum(s, 10)
    return base, n


@functools.cache
def _make_segsum():
    @functools.partial(
        pl.kernel,
        mesh=_mesh(),
        compiler_params=pltpu.CompilerParams(needs_layout_passes=False),
        out_type=jax.ShapeDtypeStruct((NREG * QN, 128), jnp.float32),
        scratch_types=[
            pltpu.VMEM((EB,), jnp.int32),
            pltpu.VMEM((EB,), jnp.int32),
            pltpu.VMEM((2 * CH,), jnp.int32),
            pltpu.VMEM((2 * CH,), jnp.int32),
            pltpu.VMEM((2 * CH,), jnp.int32),
            pltpu.VMEM((2 * CH,), jnp.int32),
            pltpu.VMEM((2 * CH,), jnp.int32),
            pltpu.VMEM((2 * CH,), jnp.int32),
            pltpu.VMEM((CH, 128), jnp.float32),
            pltpu.VMEM((CH, 128), jnp.float32),
            pltpu.VMEM((CH, 128), jnp.float32),
            pltpu.VMEM_SHARED((AR, 128), jnp.float32),
            pltpu.SemaphoreType.DMA,
            pltpu.SemaphoreType.DMA,
            pltpu.SemaphoreType.DMA,
            pltpu.SemaphoreType.DMA,
            pltpu.SemaphoreType.DMA,
            pltpu.SemaphoreType.DMA,
        ],
    )
    def _segsum_sc(table, src_g, dst_g, out, sv, dv, csrcA, cdstA, csrcB,
                   cdstB, csrcC, cdstC, rowsA, rowsB, rowsC, acc,
                   gsA, gsB, gsC, ssA, ssB, ssC):
        c = lax.axis_index("c")
        s = lax.axis_index("s")

        z16 = jnp.zeros((16,), jnp.float32)
        pos16 = lax.broadcasted_iota(jnp.int32, (16,), 0)

        nz = 4 + (s < 3).astype(jnp.int32)
        nw = 4 + (s < 2).astype(jnp.int32)
        nb = 24 + (s < 6).astype(jnp.int32)

        pairs = ((csrcA, cdstA, rowsA, gsA, ssA),
                 (csrcB, cdstB, rowsB, gsB, ssB),
                 (csrcC, cdstC, rowsC, gsC, ssC))
        first = pl.ds(0, CH)

        def _wait_gather(pair):
            csrc_o, _, rows_o, gs_o, _ = pair
            pltpu.make_async_copy(table.at[csrc_o.at[first]], rows_o,
                                  gs_o).wait()

        def _fire_scatter(pair):
            _, cdst_o, rows_o, _, ss_o = pair
            pltpu.async_copy(rows_o, acc.at[cdst_o.at[first]], ss_o,
                             add=True)

        def _wait_scatter(pair):
            _, cdst_o, rows_o, _, ss_o = pair
            pltpu.make_async_copy(rows_o, acc.at[cdst_o.at[first]],
                                  ss_o).wait()

        def _sync_drain(pair):
            _, cdst_o, rows_o, _, _ = pair
            _wait_gather(pair)
            pltpu.sync_copy(rows_o, acc.at[cdst_o.at[first]], add=True)

        def _switch3(m, f0, f1, f2):
            lax.cond(m == 0, f0, lambda: lax.cond(m == 1, f1, f2))

        def _pass(p, _):
            q = 2 * p + c
            qbase = q * QN

            def _zb(i, _):
                rowsA[i // 8, pl.ds((i % 8) * 16, 16)] = z16
                return 0
            lax.fori_loop(0, CH * 8, _zb, 0)

            def _z(k, _):
                pltpu.sync_copy(rowsA, acc.at[pl.ds((s + 16 * k) * CH, CH)])
                return 0
            lax.fori_loop(0, nz, _z, 0)
            plsc.subcore_barrier()

            def _sub_for(i):
                cur = pairs[i]
                prev = pairs[(i - 1) % 3]
                nxt = pairs[(i + 1) % 3]
                csrc_c, cdst_c, rows_c, gs_c, _ = cur

                def _sub_impl(state, sb):
                    off, fc = state
                    for v in range(CH // 16):
                        sl = pl.ds(sb + v * 16, 16)
                        local = dv[sl] - qbase
                        m = (local >= 0) & (local < QN)
                        plsc.store_compressed(csrc_c.at[pl.ds(off, 16)],
                                              sv[sl], mask=m)
                        plsc.store_compressed(cdst_c.at[pl.ds(off, 16)],
                                              local, mask=m)
                        off = off + jnp.sum(m.astype(jnp.int32))

                    def _flush(state):
                        off, fc = state
                        pltpu.async_copy(table.at[csrc_c.at[first]], rows_c,
                                         gs_c)

                        @pl.when(fc >= 1)
                        def _():
                            _wait_gather(prev)
                            _fire_scatter(prev)

                        @pl.when(fc >= 2)
                        def _():
                            _wait_scatter(nxt)
                        csrc_n, cdst_n, _, _, _ = nxt
                        for i2 in range(CH // 16):
                            t0 = csrc_c[pl.ds(CH + i2 * 16, 16)]
                            csrc_n[pl.ds(i2 * 16, 16)] = t0
                            t1 = cdst_c[pl.ds(CH + i2 * 16, 16)]
                            cdst_n[pl.ds(i2 * 16, 16)] = t1
                        return off - CH, fc + 1

                    return lax.cond(off >= CH, _flush, lambda st: st,
                                    (off, fc))
                return _sub_impl

            _subs = tuple(_sub_for(i) for i in range(3))

            def _sub(state, sb):
                off, fc = state
                m = fc % 3
                return lax.cond(
                    m == 0, lambda st: _subs[0](st, sb),
                    lambda st: lax.cond(
                        m == 1, lambda s2: _subs[1](s2, sb),
                        lambda s2: _subs[2](s2, sb), st),
                    (off, fc))

            def _step(k, state):
                eb = (s + 16 * k) * EB
                pltpu.sync_copy(src_g.at[pl.ds(eb, EB)], sv)
                pltpu.sync_copy(dst_g.at[pl.ds(eb, EB)], dv)
                return lax.fori_loop(
                    0, EB // CH, lambda j, st: _sub(st, j * CH), state)
            state = lax.fori_loop(0, nb, _step,
                                  (jnp.int32(0), jnp.int32(0)))

            def _tail(state):
                tb = 390 * EB + s * CH
                pltpu.sync_copy(src_g.at[pl.ds(tb, CH)], sv.at[pl.ds(0, CH)])
                pltpu.sync_copy(dst_g.at[pl.ds(tb, CH)], dv.at[pl.ds(0, CH)])
                return _sub(state, 0)
            off, fc = lax.cond(s < 10, _tail, lambda st: st, state)

            @pl.when(fc >= 2)
            def _():
                m2 = (fc - 2) % 3
                _switch3(m2, lambda: _wait_scatter(pairs[0]),
                         lambda: _wait_scatter(pairs[1]),
                         lambda: _wait_scatter(pairs[2]))

            @pl.when(fc >= 1)
            def _():
                m = (fc - 1) % 3
                _switch3(m, lambda: _sync_drain(pairs[0]),
                         lambda: _sync_drain(pairs[1]),
                         lambda: _sync_drain(pairs[2]))

            def _finish_for(cur):
                csrc_c, cdst_c, rows_c, gs_c, _ = cur

                def _go():
                    for v in range(CH // 16):
                        sl = pl.ds(v * 16, 16)
                        keep = (pos16 + v * 16) < off
                        csrc_c[sl] = jnp.where(keep, csrc_c[sl], 0)
                        cdst_c[sl] = jnp.where(keep, cdst_c[sl], DEAD)
                    pltpu.async_copy(table.at[csrc_c.at[first]], rows_c,
                                     gs_c).wait()
                    pltpu.sync_copy(rows_c, acc.at[cdst_c.at[first]],
                                    add=True)
                return _go

            @pl.when(off > 0)
            def _():
                _switch3(fc % 3, _finish_for(pairs[0]),
                         _finish_for(pairs[1]), _finish_for(pairs[2]))
            plsc.subcore_barrier()

            def _w(k, _):
                ch = s + 16 * k
                pltpu.sync_copy(acc.at[pl.ds(ch * CH, CH)],
                                out.at[pl.ds(qbase + ch * CH, CH)])
                return 0
            lax.fori_loop(0, nw, _w, 0)
            plsc.subcore_barrier()
            return 0
        lax.fori_loop(0, NREG // 2, _pass, 0)

    return _segsum_sc


@functools.cache
def _make_counts():
    @functools.partial(
        pl.kernel,
        mesh=_mesh(),
        compiler_params=pltpu.CompilerParams(needs_layout_passes=False),
        out_type=[
            jax.ShapeDtypeStruct((NP,), jnp.float32),
            jax.ShapeDtypeStruct((NP,), jnp.float32),
        ],
        scratch_types=[
            pltpu.VMEM((NP,), jnp.float32),
            pltpu.VMEM((EB,), jnp.int32),
            pltpu.VMEM((NPT,), jnp.float32),
            pltpu.VMEM((NPT,), jnp.float32),
            pltpu.VMEM_SHARED((16 * NP,), jnp.float32),
            pltpu.SemaphoreType.DMA,
        ],
    )
    def _counts_sc(dst_a, dst_b, out_a, out_b, acc, dblk, rbuf, res,
                   shared, sem):
        c = lax.axis_index("c")
        s = lax.axis_index("s")

        z16 = jnp.zeros((16,), jnp.float32)
        ones16 = jnp.ones((16,), jnp.float32)

        def _z(i, _):
            acc[pl.ds(i * 16, 16)] = z16
            return 0
        lax.fori_loop(0, NP // 16, _z, 0)

        nb = 24 + (s < 6).astype(jnp.int32)

        def _hist(dref):
            def _blk(b, _):
                pltpu.sync_copy(dref.at[pl.ds((s + 16 * b) * EB, EB)], dblk)

                def _vec(v, _):
                    idx = dblk[pl.ds(v * 16, 16)]
                    plsc.addupdate_scatter(acc, [idx], ones16)
                    return 0
                lax.fori_loop(0, EB // 16, _vec, 0)
                return 0
            lax.fori_loop(0, nb, _blk, 0)

            @pl.when(s < 10)
            def _():
                tb = 390 * EB + s * CH
                pltpu.sync_copy(dref.at[pl.ds(tb, CH)], dblk.at[pl.ds(0, CH)])

                def _vec(v, _):
                    idx = dblk[pl.ds(v * 16, 16)]
                    plsc.addupdate_scatter(acc, [idx], ones16)
                    return 0
                lax.fori_loop(0, CH // 16, _vec, 0)

        @pl.when(c == 0)
        def _():
            _hist(dst_a)

        @pl.when(c == 1)
        def _():
            _hist(dst_b)

        pltpu.sync_copy(acc, shared.at[pl.ds(s * NP, NP)])
        plsc.subcore_barrier()

        rb = s * NPT
        pltpu.sync_copy(shared.at[pl.ds(rb, NPT)], res)

        def _red(t, _):
            pltpu.sync_copy(shared.at[pl.ds(t * NP + rb, NPT)], rbuf)

            def _add(v, _):
                sl = pl.ds(v * 16, 16)
                res[sl] = res[sl] + rbuf[sl]
                return 0
            lax.fori_loop(0, NPT // 16, _add, 0)
            return 0
        lax.fori_loop(1, 16, _red, 0)

        @pl.when(c == 0)
        def _():
            pltpu.sync_copy(res, out_a.at[pl.ds(rb, NPT)])

        @pl.when(c == 1)
        def _():
            pltpu.sync_copy(res, out_b.at[pl.ds(rb, NPT)])

    return _counts_sc


@functools.cache
def _make_head():
    @functools.partial(
        pl.kernel,
        mesh=_mesh(),
        compiler_params=pltpu.CompilerParams(needs_layout_passes=False),
        out_type=jax.ShapeDtypeStruct((Q,), jnp.float32),
        scratch_types=[
            pltpu.VMEM((N,), jnp.float32),
            pltpu.VMEM((N,), jnp.float32),
            pltpu.VMEM((QPT,), jnp.int32),
            pltpu.VMEM((QPT,), jnp.int32),
            pltpu.VMEM((QPT,), jnp.float32),
            pltpu.SemaphoreType.DMA,
        ],
    )
    def _head_sc(p_user, p_item, eli0, eli1, out, pu, pi, i0, i1, acc, sem):
        c = lax.axis_index("c")
        s = lax.axis_index("s")
        wid = s * 2 + c
        base = wid * QPT
        pltpu.sync_copy(p_user, pu)
        pltpu.sync_copy(p_item, pi)
        pltpu.sync_copy(eli0.at[pl.ds(base, QPT)], i0)
        pltpu.sync_copy(eli1.at[pl.ds(base, QPT)], i1)

        def _g(v, _):
            sl = pl.ds(v * 16, 16)
            a = plsc.load_gather(pu, [i0[sl]])
            b = plsc.load_gather(pi, [i1[sl]])
            acc[sl] = a + b
            return 0
        lax.fori_loop(0, QPT // 16, _g, 0)
        pltpu.sync_copy(acc, out.at[pl.ds(base, QPT)])

    return _head_sc


BR = 2000
NB = N // BR


def _mm_pad(x, W):
    def body(x_ref, w_ref, o_ref):
        y = jnp.dot(x_ref[...], w_ref[...], preferred_element_type=jnp.float32)
        o_ref[...] = jnp.concatenate([y, jnp.zeros((BR, 128 - D), jnp.float32)],
                                     axis=1)
    return pl.pallas_call(
        body,
        grid=(NB,),
        in_specs=[pl.BlockSpec((BR, D), lambda i: (i, 0)),
                  pl.BlockSpec((D, D), lambda i: (0, 0))],
        out_specs=pl.BlockSpec((BR, 128), lambda i: (i, 0)),
        out_shape=jax.ShapeDtypeStruct((N, 128), jnp.float32),
    )(x, W)


def _combine(xd, W_self, b, S, cnt):
    def body(x_ref, w_ref, b_ref, s_ref, c_ref, h_ref, st_ref, acc):
        i = pl.program_id(0)
        h = (jnp.dot(x_ref[...], w_ref[...], preferred_element_type=jnp.float32)
             + s_ref[:, :D] / jnp.maximum(c_ref[...], 1.0) + b_ref[...])
        h_ref[...] = h

        @pl.when(i == 0)
        def _():
            acc[...] = jnp.zeros_like(acc)

        acc[0:1, :] += jnp.sum(h, axis=0, keepdims=True)
        acc[1:2, :] += jnp.sum(h * h, axis=0, keepdims=True)
        st_ref[...] = acc[...]

    return pl.pallas_call(
        body,
        grid=(NB,),
        in_specs=[pl.BlockSpec((BR, D), lambda i: (i, 0)),
                  pl.BlockSpec((D, D), lambda i: (0, 0)),
                  pl.BlockSpec((1, D), lambda i: (0, 0)),
                  pl.BlockSpec((BR, 128), lambda i: (i, 0)),
                  pl.BlockSpec((BR, 1), lambda i: (i, 0))],
        out_specs=[pl.BlockSpec((BR, D), lambda i: (i, 0)),
                   pl.BlockSpec((8, D), lambda i: (0, 0))],
        out_shape=[jax.ShapeDtypeStruct((N, D), jnp.float32),
                   jax.ShapeDtypeStruct((8, D), jnp.float32)],
        scratch_shapes=[pltpu.VMEM((8, D), jnp.float32)],
    )(xd, W_self, b, S, cnt)


def _bn_act(h, st, g, b, leaky):
    def body(h_ref, st_ref, g_ref, b_ref, o_ref):
        mu = st_ref[0:1, :] / N
        var = st_ref[1:2, :] / N - mu * mu
        o = (h_ref[...] - mu) / jnp.sqrt(var + 1e-5) * g_ref[...] + b_ref[...]
        if leaky:
            o = jnp.where(o >= 0, o, 0.01 * o)
        o_ref[...] = o

    return pl.pallas_call(
        body,
        grid=(NB,),
        in_specs=[pl.BlockSpec((BR, D), lambda i: (i, 0)),
                  pl.BlockSpec((8, D), lambda i: (0, 0)),
                  pl.BlockSpec((1, D), lambda i: (0, 0)),
                  pl.BlockSpec((1, D), lambda i: (0, 0))],
        out_specs=pl.BlockSpec((BR, D), lambda i: (i, 0)),
        out_shape=jax.ShapeDtypeStruct((N, D), jnp.float32),
    )(h, st, g, b)


def _bn_proj(h, st, g, b, w, bias):
    def body(h_ref, st_ref, g_ref, b_ref, w_ref, bias_ref, o_ref):
        mu = st_ref[0:1, :] / N
        var = st_ref[1:2, :] / N - mu * mu
        o = (h_ref[...] - mu) / jnp.sqrt(var + 1e-5) * g_ref[...] + b_ref[...]
        o_ref[...] = jnp.dot(o, w_ref[...],
                             preferred_element_type=jnp.float32) + bias_ref[...]

    return pl.pallas_call(
        body,
        grid=(NB,),
        in_specs=[pl.BlockSpec((BR, D), lambda i: (i, 0)),
                  pl.BlockSpec((8, D), lambda i: (0, 0)),
                  pl.BlockSpec((1, D), lambda i: (0, 0)),
                  pl.BlockSpec((1, D), lambda i: (0, 0)),
                  pl.BlockSpec((D, 1), lambda i: (0, 0)),
                  pl.BlockSpec((1, 1), lambda i: (0, 0))],
        out_specs=pl.BlockSpec((BR, 1), lambda i: (i, 0)),
        out_shape=jax.ShapeDtypeStruct((N, 1), jnp.float32),
    )(h, st, g, b, w, bias)


def kernel(x_user, x_item, edge_index_u2i, edge_index_i2u, edge_label_index_u2i,
           W_self1_u2i, W_neigh1_u2i, b1_u2i, W_self1_i2u, W_neigh1_i2u, b1_i2u,
           gamma1_user, beta1_user, gamma1_item, beta1_item,
           W_self2_u2i, W_neigh2_u2i, b2_u2i, W_self2_i2u, W_neigh2_i2u, b2_i2u,
           gamma2_user, beta2_user, gamma2_item, beta2_item,
           W_mlp, b_mlp):
    segsum = _make_segsum()
    counts = _make_counts()
    head = _make_head()

    src_u2i = edge_index_u2i[0]
    dst_u2i = edge_index_u2i[1]
    src_i2u = edge_index_i2u[0]
    dst_i2u = edge_index_i2u[1]

    cnt_item, cnt_user = counts(dst_u2i, dst_i2u)
    cnt_item = cnt_item[:N, None]
    cnt_user = cnt_user[:N, None]

    def sage(x_src, x_dst, W_neigh, W_self, b, src, dst, cnt):
        S = segsum(_mm_pad(x_src, W_neigh), src, dst)
        return _combine(x_dst, W_self, b.reshape(1, D), S[:N], cnt)

    h_item, st_i = sage(x_user, x_item, W_neigh1_u2i, W_self1_u2i, b1_u2i,
                        src_u2i, dst_u2i, cnt_item)
    h_user, st_u = sage(x_item, x_user, W_neigh1_i2u, W_self1_i2u, b1_i2u,
                        src_i2u, dst_i2u, cnt_user)

    act_user = _bn_act(h_user, st_u, gamma1_user.reshape(1, D),
                       beta1_user.reshape(1, D), leaky=True)
    act_item = _bn_act(h_item, st_i, gamma1_item.reshape(1, D),
                       beta1_item.reshape(1, D), leaky=True)

    h_item2, st_i2 = sage(act_user, act_item, W_neigh2_u2i, W_self2_u2i,
                          b2_u2i, src_u2i, dst_u2i, cnt_item)
    h_user2, st_u2 = sage(act_item, act_user, W_neigh2_i2u, W_self2_i2u,
                          b2_i2u, src_i2u, dst_i2u, cnt_user)

    p_user = _bn_proj(h_user2, st_u2, gamma2_user.reshape(1, D),
                      beta2_user.reshape(1, D), W_mlp[:D], b_mlp.reshape(1, 1))
    p_item = _bn_proj(h_item2, st_i2, gamma2_item.reshape(1, D),
                      beta2_item.reshape(1, D), W_mlp[D:],
                      jnp.zeros((1, 1), jnp.float32))

    pred = head(p_user.reshape(N), p_item.reshape(N),
                edge_label_index_u2i[0], edge_label_index_u2i[1])
    return pred


# speedup vs baseline: 1.2083x; 1.1076x over previous
import functools

import jax
import jax.numpy as jnp
from jax import lax
from jax.experimental import pallas as pl
from jax.experimental.pallas import tpu as pltpu
from jax.experimental.pallas import tpu_sc as plsc

N = 50000
E = 800000
D = 64
Q = 8192

CH = 128
EB = 2048
TCH = E // CH
QN = 8448
NREG = 6
DEAD = QN
AR = QN + CH
NP = 51200
NPT = NP // 16

QPT = Q // 32


def _mesh():
    return plsc.VectorSubcoreMesh(core_axis_name="c", subcore_axis_name="s")


def _chunk_range(s):
    n = 390 + (s < 10).astype(jnp.int32)
    base = s * 390 + jnp.minimum(s, 10)
    return base, n


@functools.cache
def _make_segsum():
    @functools.partial(
        pl.kernel,
        mesh=_mesh(),
        compiler_params=pltpu.CompilerParams(needs_layout_passes=False),
        out_type=jax.ShapeDtypeStruct((NREG * QN, 128), jnp.float32),
        scratch_types=[
            pltpu.VMEM((EB,), jnp.int32),
            pltpu.VMEM((EB,), jnp.int32),
            pltpu.VMEM((EB,), jnp.int32),
            pltpu.VMEM((EB,), jnp.int32),
            pltpu.VMEM((2 * CH,), jnp.int32),
            pltpu.VMEM((2 * CH,), jnp.int32),
            pltpu.VMEM((2 * CH,), jnp.int32),
            pltpu.VMEM((2 * CH,), jnp.int32),
            pltpu.VMEM((2 * CH,), jnp.int32),
            pltpu.VMEM((2 * CH,), jnp.int32),
            pltpu.VMEM((CH, 128), jnp.float32),
            pltpu.VMEM((CH, 128), jnp.float32),
            pltpu.VMEM((CH, 128), jnp.float32),
            pltpu.VMEM_SHARED((AR, 128), jnp.float32),
            pltpu.SemaphoreType.DMA,
            pltpu.SemaphoreType.DMA,
            pltpu.SemaphoreType.DMA,
            pltpu.SemaphoreType.DMA,
            pltpu.SemaphoreType.DMA,
            pltpu.SemaphoreType.DMA,
            pltpu.SemaphoreType.DMA,
            pltpu.SemaphoreType.DMA,
        ],
    )
    def _segsum_sc(table, src_g, dst_g, out, sv0, dv0, sv1, dv1,
                   csrcA, cdstA, csrcB, cdstB, csrcC, cdstC,
                   rowsA, rowsB, rowsC, acc,
                   gsA, gsB, gsC, ssA, ssB, ssC, isem0, isem1):
        c = lax.axis_index("c")
        s = lax.axis_index("s")

        z16 = jnp.zeros((16,), jnp.float32)
        pos16 = lax.broadcasted_iota(jnp.int32, (16,), 0)

        nz = 4 + (s < 3).astype(jnp.int32)
        nw = 4 + (s < 2).astype(jnp.int32)
        nb = 24 + (s < 6).astype(jnp.int32)

        pairs = ((csrcA, cdstA, rowsA, gsA, ssA),
                 (csrcB, cdstB, rowsB, gsB, ssB),
                 (csrcC, cdstC, rowsC, gsC, ssC))
        first = pl.ds(0, CH)

        def _wait_gather(pair):
            csrc_o, _, rows_o, gs_o, _ = pair
            pltpu.make_async_copy(table.at[csrc_o.at[first]], rows_o,
                                  gs_o).wait()

        def _fire_scatter(pair):
            _, cdst_o, rows_o, _, ss_o = pair
            pltpu.async_copy(rows_o, acc.at[cdst_o.at[first]], ss_o,
                             add=True)

        def _wait_scatter(pair):
            _, cdst_o, rows_o, _, ss_o = pair
            pltpu.make_async_copy(rows_o, acc.at[cdst_o.at[first]],
                                  ss_o).wait()

        def _sync_drain(pair):
            _, cdst_o, rows_o, _, _ = pair
            _wait_gather(pair)
            pltpu.sync_copy(rows_o, acc.at[cdst_o.at[first]], add=True)

        def _switch3(m, f0, f1, f2):
            lax.cond(m == 0, f0, lambda: lax.cond(m == 1, f1, f2))

        def _pass(p, _):
            q = 2 * p + c
            qbase = q * QN

            def _zb(i, _):
                rowsA[i // 8, pl.ds((i % 8) * 16, 16)] = z16
                return 0
            lax.fori_loop(0, CH * 8, _zb, 0)

            def _z(k, _):
                pltpu.sync_copy(rowsA, acc.at[pl.ds((s + 16 * k) * CH, CH)])
                return 0
            lax.fori_loop(0, nz, _z, 0)
            plsc.subcore_barrier()

            def _sub_for(i, svb, dvb):
                cur = pairs[i]
                prev = pairs[(i - 1) % 3]
                nxt = pairs[(i + 1) % 3]
                csrc_c, cdst_c, rows_c, gs_c, _ = cur

                def _sub_impl(state, sb):
                    off, fc = state
                    for v in range(CH // 16):
                        sl = pl.ds(sb + v * 16, 16)
                        local = dvb[sl] - qbase
                        m = (local >= 0) & (local < QN)
                        plsc.store_compressed(csrc_c.at[pl.ds(off, 16)],
                                              svb[sl], mask=m)
                        plsc.store_compressed(cdst_c.at[pl.ds(off, 16)],
                                              local, mask=m)
                        off = off + jnp.sum(m.astype(jnp.int32))

                    def _flush(state):
                        off, fc = state
                        pltpu.async_copy(table.at[csrc_c.at[first]], rows_c,
                                         gs_c)

                        @pl.when(fc >= 1)
                        def _():
                            _wait_gather(prev)
                            _fire_scatter(prev)

                        @pl.when(fc >= 2)
                        def _():
                            _wait_scatter(nxt)
                        csrc_n, cdst_n, _, _, _ = nxt
                        for i2 in range(CH // 16):
                            t0 = csrc_c[pl.ds(CH + i2 * 16, 16)]
                            csrc_n[pl.ds(i2 * 16, 16)] = t0
                            t1 = cdst_c[pl.ds(CH + i2 * 16, 16)]
                            cdst_n[pl.ds(i2 * 16, 16)] = t1
                        return off - CH, fc + 1

                    return lax.cond(off >= CH, _flush, lambda st: st,
                                    (off, fc))
                return _sub_impl

            ibufs = ((sv0, dv0, isem0), (sv1, dv1, isem1))
            _subs2 = tuple(
                tuple(_sub_for(i, svb, dvb) for i in range(3))
                for (svb, dvb, _) in ibufs)

            def _sub_on(bi, state, sb):
                subs = _subs2[bi]
                off, fc = state
                m = fc % 3
                return lax.cond(
                    m == 0, lambda st: subs[0](st, sb),
                    lambda st: lax.cond(
                        m == 1, lambda s2: subs[1](s2, sb),
                        lambda s2: subs[2](s2, sb), st),
                    (off, fc))

            def _fire_iload(k, bi):
                svb, dvb, isb = ibufs[bi]
                eb = (s + 16 * k) * EB
                pltpu.async_copy(src_g.at[pl.ds(eb, EB)], svb, isb)
                pltpu.async_copy(dst_g.at[pl.ds(eb, EB)], dvb, isb)

            def _wait_iload(k, bi):
                svb, dvb, isb = ibufs[bi]
                eb = (s + 16 * k) * EB
                pltpu.make_async_copy(src_g.at[pl.ds(eb, EB)], svb,
                                      isb).wait()
                pltpu.make_async_copy(dst_g.at[pl.ds(eb, EB)], dvb,
                                      isb).wait()

            _fire_iload(0, 0)

            def _step_on(bi, k, state):
                _wait_iload(k, bi)

                @pl.when(k + 1 < nb)
                def _():
                    _fire_iload(k + 1, 1 - bi)
                return lax.fori_loop(
                    0, EB // CH, lambda j, st: _sub_on(bi, st, j * CH), state)

            def _step(k, state):
                return lax.cond(k % 2 == 0,
                                lambda st: _step_on(0, k, st),
                                lambda st: _step_on(1, k, st),
                                state)
            state = lax.fori_loop(0, nb, _step,
                                  (jnp.int32(0), jnp.int32(0)))

            def _tail(state):
                tb = 390 * EB + s * CH
                pltpu.sync_copy(src_g.at[pl.ds(tb, CH)], sv0.at[pl.ds(0, CH)])
                pltpu.sync_copy(dst_g.at[pl.ds(tb, CH)], dv0.at[pl.ds(0, CH)])
                return _sub_on(0, state, 0)
            off, fc = lax.cond(s < 10, _tail, lambda st: st, state)

            @pl.when(fc >= 2)
            def _():
                m2 = (fc - 2) % 3
                _switch3(m2, lambda: _wait_scatter(pairs[0]),
                         lambda: _wait_scatter(pairs[1]),
                         lambda: _wait_scatter(pairs[2]))

            @pl.when(fc >= 1)
            def _():
                m = (fc - 1) % 3
                _switch3(m, lambda: _sync_drain(pairs[0]),
                         lambda: _sync_drain(pairs[1]),
                         lambda: _sync_drain(pairs[2]))

            def _finish_for(cur):
                csrc_c, cdst_c, rows_c, gs_c, _ = cur

                def _go():
                    for v in range(CH // 16):
                        sl = pl.ds(v * 16, 16)
                        keep = (pos16 + v * 16) < off
                        csrc_c[sl] = jnp.where(keep, csrc_c[sl], 0)
                        cdst_c[sl] = jnp.where(keep, cdst_c[sl], DEAD)
                    pltpu.async_copy(table.at[csrc_c.at[first]], rows_c,
                                     gs_c).wait()
                    pltpu.sync_copy(rows_c, acc.at[cdst_c.at[first]],
                                    add=True)
                return _go

            @pl.when(off > 0)
            def _():
                _switch3(fc % 3, _finish_for(pairs[0]),
                         _finish_for(pairs[1]), _finish_for(pairs[2]))
            plsc.subcore_barrier()

            def _w(k, _):
                ch = s + 16 * k
                pltpu.sync_copy(acc.at[pl.ds(ch * CH, CH)],
                                out.at[pl.ds(qbase + ch * CH, CH)])
                return 0
            lax.fori_loop(0, nw, _w, 0)
            plsc.subcore_barrier()
            return 0
        lax.fori_loop(0, NREG // 2, _pass, 0)

    return _segsum_sc


@functools.cache
def _make_counts():
    @functools.partial(
        pl.kernel,
        mesh=_mesh(),
        compiler_params=pltpu.CompilerParams(needs_layout_passes=False),
        out_type=[
            jax.ShapeDtypeStruct((NP,), jnp.float32),
            jax.ShapeDtypeStruct((NP,), jnp.float32),
        ],
        scratch_types=[
            pltpu.VMEM((NP,), jnp.float32),
            pltpu.VMEM((EB,), jnp.int32),
            pltpu.VMEM((NPT,), jnp.float32),
            pltpu.VMEM((NPT,), jnp.float32),
            pltpu.VMEM_SHARED((16 * NP,), jnp.float32),
            pltpu.SemaphoreType.DMA,
        ],
    )
    def _counts_sc(dst_a, dst_b, out_a, out_b, acc, dblk, rbuf, res,
                   shared, sem):
        c = lax.axis_index("c")
        s = lax.axis_index("s")

        z16 = jnp.zeros((16,), jnp.float32)
        ones16 = jnp.ones((16,), jnp.float32)

        def _z(i, _):
            acc[pl.ds(i * 16, 16)] = z16
            return 0
        lax.fori_loop(0, NP // 16, _z, 0)

        nb = 24 + (s < 6).astype(jnp.int32)

        def _hist(dref):
            def _blk(b, _):
                pltpu.sync_copy(dref.at[pl.ds((s + 16 * b) * EB, EB)], dblk)

                def _vec(v, _):
                    idx = dblk[pl.ds(v * 16, 16)]
                    plsc.addupdate_scatter(acc, [idx], ones16)
                    return 0
                lax.fori_loop(0, EB // 16, _vec, 0)
                return 0
            lax.fori_loop(0, nb, _blk, 0)

            @pl.when(s < 10)
            def _():
                tb = 390 * EB + s * CH
                pltpu.sync_copy(dref.at[pl.ds(tb, CH)], dblk.at[pl.ds(0, CH)])

                def _vec(v, _):
                    idx = dblk[pl.ds(v * 16, 16)]
                    plsc.addupdate_scatter(acc, [idx], ones16)
                    return 0
                lax.fori_loop(0, CH // 16, _vec, 0)

        @pl.when(c == 0)
        def _():
            _hist(dst_a)

        @pl.when(c == 1)
        def _():
            _hist(dst_b)

        pltpu.sync_copy(acc, shared.at[pl.ds(s * NP, NP)])
        plsc.subcore_barrier()

        rb = s * NPT
        pltpu.sync_copy(shared.at[pl.ds(rb, NPT)], res)

        def _red(t, _):
            pltpu.sync_copy(shared.at[pl.ds(t * NP + rb, NPT)], rbuf)

            def _add(v, _):
                sl = pl.ds(v * 16, 16)
                res[sl] = res[sl] + rbuf[sl]
                return 0
            lax.fori_loop(0, NPT // 16, _add, 0)
            return 0
        lax.fori_loop(1, 16, _red, 0)

        @pl.when(c == 0)
        def _():
            pltpu.sync_copy(res, out_a.at[pl.ds(rb, NPT)])

        @pl.when(c == 1)
        def _():
            pltpu.sync_copy(res, out_b.at[pl.ds(rb, NPT)])

    return _counts_sc


@functools.cache
def _make_head():
    @functools.partial(
        pl.kernel,
        mesh=_mesh(),
        compiler_params=pltpu.CompilerParams(needs_layout_passes=False),
        out_type=jax.ShapeDtypeStruct((Q,), jnp.float32),
        scratch_types=[
            pltpu.VMEM((N,), jnp.float32),
            pltpu.VMEM((N,), jnp.float32),
            pltpu.VMEM((QPT,), jnp.int32),
            pltpu.VMEM((QPT,), jnp.int32),
            pltpu.VMEM((QPT,), jnp.float32),
            pltpu.SemaphoreType.DMA,
        ],
    )
    def _head_sc(p_user, p_item, eli0, eli1, out, pu, pi, i0, i1, acc, sem):
        c = lax.axis_index("c")
        s = lax.axis_index("s")
        wid = s * 2 + c
        base = wid * QPT
        pltpu.sync_copy(p_user, pu)
        pltpu.sync_copy(p_item, pi)
        pltpu.sync_copy(eli0.at[pl.ds(base, QPT)], i0)
        pltpu.sync_copy(eli1.at[pl.ds(base, QPT)], i1)

        def _g(v, _):
            sl = pl.ds(v * 16, 16)
            a = plsc.load_gather(pu, [i0[sl]])
            b = plsc.load_gather(pi, [i1[sl]])
            acc[sl] = a + b
            return 0
        lax.fori_loop(0, QPT // 16, _g, 0)
        pltpu.sync_copy(acc, out.at[pl.ds(base, QPT)])

    return _head_sc


BR = 2000
NB = N // BR


def _mm_pad(x, W):
    def body(x_ref, w_ref, o_ref):
        y = jnp.dot(x_ref[...], w_ref[...], preferred_element_type=jnp.float32)
        o_ref[...] = jnp.concatenate([y, jnp.zeros((BR, 128 - D), jnp.float32)],
                                     axis=1)
    return pl.pallas_call(
        body,
        grid=(NB,),
        in_specs=[pl.BlockSpec((BR, D), lambda i: (i, 0)),
                  pl.BlockSpec((D, D), lambda i: (0, 0))],
        out_specs=pl.BlockSpec((BR, 128), lambda i: (i, 0)),
        out_shape=jax.ShapeDtypeStruct((N, 128), jnp.float32),
    )(x, W)


def _combine(xd, W_self, b, S, cnt):
    def body(x_ref, w_ref, b_ref, s_ref, c_ref, h_ref, st_ref, acc):
        i = pl.program_id(0)
        h = (jnp.dot(x_ref[...], w_ref[...], preferred_element_type=jnp.float32)
             + s_ref[:, :D] / jnp.maximum(c_ref[...], 1.0) + b_ref[...])
        h_ref[...] = h

        @pl.when(i == 0)
        def _():
            acc[...] = jnp.zeros_like(acc)

        acc[0:1, :] += jnp.sum(h, axis=0, keepdims=True)
        acc[1:2, :] += jnp.sum(h * h, axis=0, keepdims=True)
        st_ref[...] = acc[...]

    return pl.pallas_call(
        body,
        grid=(NB,),
        in_specs=[pl.BlockSpec((BR, D), lambda i: (i, 0)),
                  pl.BlockSpec((D, D), lambda i: (0, 0)),
                  pl.BlockSpec((1, D), lambda i: (0, 0)),
                  pl.BlockSpec((BR, 128), lambda i: (i, 0)),
                  pl.BlockSpec((BR, 1), lambda i: (i, 0))],
        out_specs=[pl.BlockSpec((BR, D), lambda i: (i, 0)),
                   pl.BlockSpec((8, D), lambda i: (0, 0))],
        out_shape=[jax.ShapeDtypeStruct((N, D), jnp.float32),
                   jax.ShapeDtypeStruct((8, D), jnp.float32)],
        scratch_shapes=[pltpu.VMEM((8, D), jnp.float32)],
    )(xd, W_self, b, S, cnt)


def _bn_act(h, st, g, b, leaky):
    def body(h_ref, st_ref, g_ref, b_ref, o_ref):
        mu = st_ref[0:1, :] / N
        var = st_ref[1:2, :] / N - mu * mu
        o = (h_ref[...] - mu) / jnp.sqrt(var + 1e-5) * g_ref[...] + b_ref[...]
        if leaky:
            o = jnp.where(o >= 0, o, 0.01 * o)
        o_ref[...] = o

    return pl.pallas_call(
        body,
        grid=(NB,),
        in_specs=[pl.BlockSpec((BR, D), lambda i: (i, 0)),
                  pl.BlockSpec((8, D), lambda i: (0, 0)),
                  pl.BlockSpec((1, D), lambda i: (0, 0)),
                  pl.BlockSpec((1, D), lambda i: (0, 0))],
        out_specs=pl.BlockSpec((BR, D), lambda i: (i, 0)),
        out_shape=jax.ShapeDtypeStruct((N, D), jnp.float32),
    )(h, st, g, b)


def _bn_proj(h, st, g, b, w, bias):
    def body(h_ref, st_ref, g_ref, b_ref, w_ref, bias_ref, o_ref):
        mu = st_ref[0:1, :] / N
        var = st_ref[1:2, :] / N - mu * mu
        o = (h_ref[...] - mu) / jnp.sqrt(var + 1e-5) * g_ref[...] + b_ref[...]
        o_ref[...] = jnp.dot(o, w_ref[...],
                             preferred_element_type=jnp.float32) + bias_ref[...]

    return pl.pallas_call(
        body,
        grid=(NB,),
        in_specs=[pl.BlockSpec((BR, D), lambda i: (i, 0)),
                  pl.BlockSpec((8, D), lambda i: (0, 0)),
                  pl.BlockSpec((1, D), lambda i: (0, 0)),
                  pl.BlockSpec((1, D), lambda i: (0, 0)),
                  pl.BlockSpec((D, 1), lambda i: (0, 0)),
                  pl.BlockSpec((1, 1), lambda i: (0, 0))],
        out_specs=pl.BlockSpec((BR, 1), lambda i: (i, 0)),
        out_shape=jax.ShapeDtypeStruct((N, 1), jnp.float32),
    )(h, st, g, b, w, bias)


def kernel(x_user, x_item, edge_index_u2i, edge_index_i2u, edge_label_index_u2i,
           W_self1_u2i, W_neigh1_u2i, b1_u2i, W_self1_i2u, W_neigh1_i2u, b1_i2u,
           gamma1_user, beta1_user, gamma1_item, beta1_item,
           W_self2_u2i, W_neigh2_u2i, b2_u2i, W_self2_i2u, W_neigh2_i2u, b2_i2u,
           gamma2_user, beta2_user, gamma2_item, beta2_item,
           W_mlp, b_mlp):
    segsum = _make_segsum()
    counts = _make_counts()
    head = _make_head()

    src_u2i = edge_index_u2i[0]
    dst_u2i = edge_index_u2i[1]
    src_i2u = edge_index_i2u[0]
    dst_i2u = edge_index_i2u[1]

    cnt_item, cnt_user = counts(dst_u2i, dst_i2u)
    cnt_item = cnt_item[:N, None]
    cnt_user = cnt_user[:N, None]

    def sage(x_src, x_dst, W_neigh, W_self, b, src, dst, cnt):
        S = segsum(_mm_pad(x_src, W_neigh), src, dst)
        return _combine(x_dst, W_self, b.reshape(1, D), S[:N], cnt)

    h_item, st_i = sage(x_user, x_item, W_neigh1_u2i, W_self1_u2i, b1_u2i,
                        src_u2i, dst_u2i, cnt_item)
    h_user, st_u = sage(x_item, x_user, W_neigh1_i2u, W_self1_i2u, b1_i2u,
                        src_i2u, dst_i2u, cnt_user)

    act_user = _bn_act(h_user, st_u, gamma1_user.reshape(1, D),
                       beta1_user.reshape(1, D), leaky=True)
    act_item = _bn_act(h_item, st_i, gamma1_item.reshape(1, D),
                       beta1_item.reshape(1, D), leaky=True)

    h_item2, st_i2 = sage(act_user, act_item, W_neigh2_u2i, W_self2_u2i,
                          b2_u2i, src_u2i, dst_u2i, cnt_item)
    h_user2, st_u2 = sage(act_item, act_user, W_neigh2_i2u, W_self2_i2u,
                          b2_i2u, src_i2u, dst_i2u, cnt_user)

    p_user = _bn_proj(h_user2, st_u2, gamma2_user.reshape(1, D),
                      beta2_user.reshape(1, D), W_mlp[:D], b_mlp.reshape(1, 1))
    p_item = _bn_proj(h_item2, st_i2, gamma2_item.reshape(1, D),
                      beta2_item.reshape(1, D), W_mlp[D:],
                      jnp.zeros((1, 1), jnp.float32))

    pred = head(p_user.reshape(N), p_item.reshape(N),
                edge_label_index_u2i[0], edge_label_index_u2i[1])
    return pred
